# baseline (speedup 1.0000x reference)
import jax
import jax.numpy as jnp
from jax.experimental import pallas as pl
from jax.experimental.pallas import tpu as pltpu
from jax.experimental.pallas import tpu_sc as plsc

_VOCAB = 100000
_EMBED = 64
_BATCH = 1024

_V_TILE = 2048

_SC_CORES = 2
_SC_SUBCORES = 16
_SC_WORKERS = _SC_CORES * _SC_SUBCORES
_B_PER_W = _BATCH // _SC_WORKERS


_B_PER_SCS = _BATCH // _SC_CORES


def _sc_gather(table, x):
    mesh = plsc.ScalarSubcoreMesh(axis_name="core", num_cores=_SC_CORES)

    @pl.kernel(
        out_type=jax.ShapeDtypeStruct((_BATCH, _EMBED), table.dtype),
        mesh=mesh,
        scratch_types=[
            pltpu.SMEM((_B_PER_SCS,), jnp.int32),
            pltpu.SemaphoreType.DMA,
            pltpu.SemaphoreType.DMA,
        ],
    )
    def gather_kernel(tbl_hbm, i_hbm, o_hbm, idx_s, isem, sem):
        c = jax.lax.axis_index("core")
        base = c * _B_PER_SCS
        pltpu.async_copy(i_hbm.at[pl.ds(base, _B_PER_SCS)], idx_s, isem).wait()

        @pl.loop(0, _B_PER_SCS)
        def _(i):
            pltpu.async_copy(tbl_hbm.at[idx_s[i]], o_hbm.at[base + i], sem)

        @pl.loop(0, _B_PER_SCS)
        def _(i):
            pltpu.make_async_copy(tbl_hbm.at[0], o_hbm.at[base + i], sem).wait()

    return gather_kernel(table, x)


_N_CORES = 2
_STEPS = 25
_TILES = 49
_LAST = _TILES - 1
_TAIL = _VOCAB - _LAST * _V_TILE
_NBUF = 4


def _mm_body(e_ref, w_ref, b_ref, o_hbm, acc_ref, tacc_ref, sems, tsem):
    i = pl.program_id(0)
    j = pl.program_id(1)
    t = i * _STEPS + j
    slot = jax.lax.rem(j, _NBUF)
    col = t * _V_TILE

    def full_copy(s, tt):
        return pltpu.make_async_copy(
            acc_ref.at[s], o_hbm.at[:, pl.ds(tt * _V_TILE, _V_TILE)], sems.at[s]
        )

    def tail_copy():
        return pltpu.make_async_copy(
            tacc_ref,
            o_hbm.at[:, pl.ds(_LAST * _V_TILE, _TAIL)],
            tsem,
        )

    @pl.when(jnp.logical_and(j >= _NBUF, t <= _LAST))
    def _():
        full_copy(slot, t - _NBUF).wait()

    e_bf = e_ref[...].astype(jnp.bfloat16)

    @pl.when(t < _LAST)
    def _():
        acc_ref[slot] = jax.lax.dot_general(
            e_bf, w_ref[...],
            (((1,), (1,)), ((), ())),
            preferred_element_type=jnp.float32,
        ) + b_ref[...]
        full_copy(slot, t).start()

    @pl.when(t == _LAST)
    def _():
        full = jax.lax.dot_general(
            e_bf, w_ref[...],
            (((1,), (1,)), ((), ())),
            preferred_element_type=jnp.float32,
        ) + b_ref[...]
        tacc_ref[...] = jax.lax.slice(full, (0, 0), (_BATCH, _TAIL))
        tail_copy().start()

    @pl.when(jnp.logical_and(j == _STEPS - 1, i == 0))
    def _():
        for k in range(_NBUF):
            full_copy(k, 0).wait()

    @pl.when(jnp.logical_and(j == _STEPS - 1, i == 1))
    def _():
        for k in range(_NBUF - 1):
            full_copy(k, 0).wait()
        tail_copy().wait()


def _tc_project(e, W_bf, b):
    b2 = b.reshape(1, _VOCAB)
    clamp = lambda t: jnp.minimum(t, _LAST)
    return pl.pallas_call(
        _mm_body,
        grid=(_N_CORES, _STEPS),
        in_specs=[
            pl.BlockSpec((_BATCH, _EMBED), lambda i, j: (0, 0)),
            pl.BlockSpec((_V_TILE, _EMBED),
                         lambda i, j: (clamp(i * _STEPS + j), 0)),
            pl.BlockSpec((1, _V_TILE),
                         lambda i, j: (0, clamp(i * _STEPS + j))),
        ],
        out_specs=pl.BlockSpec(memory_space=pltpu.MemorySpace.HBM),
        out_shape=jax.ShapeDtypeStruct((_BATCH, _VOCAB), jnp.float32),
        scratch_shapes=[
            pltpu.VMEM((_NBUF, _BATCH, _V_TILE), jnp.float32),
            pltpu.VMEM((_BATCH, _TAIL), jnp.float32),
            pltpu.SemaphoreType.DMA((_NBUF,)),
            pltpu.SemaphoreType.DMA,
        ],
        compiler_params=pltpu.CompilerParams(
            dimension_semantics=("parallel", "arbitrary"),
        ),
    )(e, W_bf, b2)


def kernel(x, table, W, b):
    xi = x.astype(jnp.int32)
    e = _sc_gather(table, xi)
    W_bf = W.astype(jnp.bfloat16)
    logits = _tc_project(e, W_bf, b)
    return (logits, e)


# speedup vs baseline: 2.8908x; 2.8908x over previous
import jax
import jax.numpy as jnp
from jax.experimental import pallas as pl
from jax.experimental.pallas import tpu as pltpu
from jax.experimental.pallas import tpu_sc as plsc

_VOCAB = 100000
_EMBED = 64
_BATCH = 1024

_V_TILE = 2048

_SC_CORES = 2
_SC_SUBCORES = 16
_SC_WORKERS = _SC_CORES * _SC_SUBCORES
_B_PER_W = _BATCH // _SC_WORKERS


_B_PER_SCS = _BATCH // _SC_CORES


def _sc_gather(table, x):
    mesh = plsc.ScalarSubcoreMesh(axis_name="core", num_cores=_SC_CORES)

    @pl.kernel(
        out_type=jax.ShapeDtypeStruct((_BATCH, _EMBED), table.dtype),
        mesh=mesh,
        scratch_types=[
            pltpu.SMEM((_B_PER_SCS,), jnp.int32),
            pltpu.SemaphoreType.DMA,
            pltpu.SemaphoreType.DMA,
        ],
    )
    def gather_kernel(tbl_hbm, i_hbm, o_hbm, idx_s, isem, sem):
        c = jax.lax.axis_index("core")
        base = c * _B_PER_SCS
        pltpu.async_copy(i_hbm.at[pl.ds(base, _B_PER_SCS)], idx_s, isem).wait()

        @pl.loop(0, _B_PER_SCS)
        def _(i):
            pltpu.async_copy(tbl_hbm.at[idx_s[i]], o_hbm.at[base + i], sem)

        @pl.loop(0, _B_PER_SCS)
        def _(i):
            pltpu.make_async_copy(tbl_hbm.at[0], o_hbm.at[base + i], sem).wait()

    return gather_kernel(table, x)


_TILES = 49
_LAST = _TILES - 1
_TAIL = _VOCAB - _LAST * _V_TILE
_NBUF = 4


def _mm_body(e_ref, wt_ref, o_hbm, acc_ref, sems):
    j = pl.program_id(0)
    slot = jax.lax.rem(j, _NBUF)

    def out_copy(s, tt, rows):
        return pltpu.make_async_copy(
            acc_ref.at[s, pl.ds(0, rows)],
            o_hbm.at[pl.ds(tt * _V_TILE, rows)],
            sems.at[s],
        )

    @pl.when(j >= _NBUF)
    def _():
        out_copy(slot, j - _NBUF, _V_TILE).wait()

    acc_ref[slot] = jax.lax.dot_general(
        wt_ref[...], e_ref[...].astype(jnp.bfloat16),
        (((0,), (1,)), ((), ())),
        preferred_element_type=jnp.float32,
    )

    @pl.when(j < _LAST)
    def _():
        out_copy(slot, j, _V_TILE).start()

    @pl.when(j == _LAST)
    def _():
        out_copy(slot, j, _TAIL).start()
        for k in range(_NBUF - 1):
            out_copy((_LAST - _NBUF + 1 + k) % _NBUF,
                     0, _V_TILE).wait()
        out_copy(slot, 0, _TAIL).wait()


def _tc_project(e_aug, Wt_aug):
    return pl.pallas_call(
        _mm_body,
        grid=(_TILES,),
        in_specs=[
            pl.BlockSpec((_BATCH, _EMBED + 1), lambda j: (0, 0)),
            pl.BlockSpec((_EMBED + 1, _V_TILE), lambda j: (0, j)),
        ],
        out_specs=pl.BlockSpec(memory_space=pltpu.MemorySpace.HBM),
        out_shape=jax.ShapeDtypeStruct((_VOCAB, _BATCH), jnp.float32),
        scratch_shapes=[
            pltpu.VMEM((_NBUF, _V_TILE, _BATCH), jnp.float32),
            pltpu.SemaphoreType.DMA((_NBUF,)),
        ],
        compiler_params=pltpu.CompilerParams(
            dimension_semantics=("arbitrary",),
        ),
    )(e_aug, Wt_aug)


def kernel(x, table, W, b):
    xi = x.astype(jnp.int32)
    e = _sc_gather(table, xi)
    Wt_aug = jnp.concatenate([W.T, b[None, :]], axis=0).astype(jnp.bfloat16)
    e_aug = jnp.concatenate(
        [e, jnp.ones((_BATCH, 1), jnp.float32)], axis=1)
    logitsT = _tc_project(e_aug, Wt_aug)
    return (logitsT.T, e)
